# baseline (speedup 1.0000x reference)
import functools
import math

import jax
import jax.numpy as jnp
from jax import lax
from jax.experimental import pallas as pl
from jax.experimental.pallas import tpu as pltpu
from jax.experimental.pallas import tpu_sc as plsc

_VOCAB = 100000
_DIM = 64
_B = 16384
_X_MAX = 100.0
_ALPHA = 0.75
_LOG_XMAX = math.log(_X_MAX)

_NC, _NS, _L = 2, 16, 16
_NW = _NC * _NS
_BPW = _B // _NW

_mesh = plsc.VectorSubcoreMesh(core_axis_name="c", subcore_axis_name="s")


@functools.partial(
    pl.kernel,
    out_type=jax.ShapeDtypeStruct((_B,), jnp.float32),
    mesh=_mesh,
    compiler_params=pltpu.CompilerParams(needs_layout_passes=False,
                                         use_tc_tiling_on_sc=False),
    scratch_types=[
        pltpu.VMEM((_BPW,), jnp.int32),
        pltpu.VMEM((_BPW,), jnp.int32),
        pltpu.VMEM((_BPW, _DIM), jnp.float32),
        pltpu.VMEM((_BPW, _DIM), jnp.float32),
        pltpu.VMEM((_BPW,), jnp.float32),
        pltpu.VMEM((_BPW,), jnp.float32),
        pltpu.VMEM((_BPW,), jnp.float32),
        pltpu.SemaphoreType.DMA,
    ],
)
def _sc_dot(cidx_hbm, pidx_hbm, cemb_hbm, cbias_hbm, pemb_hbm, pbias_hbm,
            out_hbm, cidx_v, pidx_v, ce_v, pe_v, cb_v, pb_v, s_v, sem):
    wid = lax.axis_index("s") * _NC + lax.axis_index("c")
    base = wid * _BPW
    pltpu.sync_copy(cidx_hbm.at[pl.ds(base, _BPW)], cidx_v)
    pltpu.sync_copy(pidx_hbm.at[pl.ds(base, _BPW)], pidx_v)
    c1 = pltpu.async_copy(cemb_hbm.at[cidx_v], ce_v, sem)
    c2 = pltpu.async_copy(pemb_hbm.at[pidx_v], pe_v, sem)
    c3 = pltpu.async_copy(cbias_hbm.at[cidx_v], cb_v, sem)
    c4 = pltpu.async_copy(pbias_hbm.at[pidx_v], pb_v, sem)
    c1.wait()
    c2.wait()
    c3.wait()
    c4.wait()

    iot = lax.iota(jnp.int32, _L)

    def group(g, carry):
        rows = g * _L + iot
        acc0 = cb_v[pl.ds(g * _L, _L)] + pb_v[pl.ds(g * _L, _L)]

        def dstep(d, acc):
            cols = jnp.full((_L,), d, jnp.int32)
            return acc + (plsc.load_gather(ce_v, [rows, cols]) *
                          plsc.load_gather(pe_v, [rows, cols]))

        acc = lax.fori_loop(0, _DIM, dstep, acc0)
        s_v[pl.ds(g * _L, _L)] = acc
        return carry

    lax.fori_loop(0, _BPW // _L, group, 0)
    pltpu.sync_copy(s_v, out_hbm.at[pl.ds(base, _BPW)])


def _loss_body(s_ref, lab_ref, out_ref):
    lab = lab_ref[...]
    ll = jnp.log(lab)
    w = jnp.minimum(jnp.exp(_ALPHA * (ll - _LOG_XMAX)), 1.0)
    diff = s_ref[...] - ll
    out_ref[0, 0] = jnp.sum(w * diff * diff) * (1.0 / _B)


_loss_call = pl.pallas_call(
    _loss_body,
    out_shape=jax.ShapeDtypeStruct((1, 1), jnp.float32),
    in_specs=[
        pl.BlockSpec(memory_space=pltpu.VMEM),
        pl.BlockSpec(memory_space=pltpu.VMEM),
    ],
    out_specs=pl.BlockSpec(memory_space=pltpu.SMEM),
)


def kernel(c_data, p_data, labels, c_embed, c_bias, p_embed, p_bias):
    s = _sc_dot(c_data.astype(jnp.int32), p_data.astype(jnp.int32),
                c_embed, c_bias.reshape(-1), p_embed, p_bias.reshape(-1))
    out = _loss_call(s.reshape(128, 128), labels.reshape(128, 128))
    return out[0, 0]
